# baseline (speedup 1.0000x reference)
import functools
import math

import jax
import jax.numpy as jnp
from jax import lax
from jax.experimental import pallas as pl
from jax.experimental.pallas import tpu as pltpu
from jax.experimental.pallas import tpu_sc as plsc

_N_EXP = 8
_TOP_K = 2
_CAP_FACTOR = 2.0
_MIN_CAPACITY = 4

_NC = 2
_NS = 16
_NW = _NC * _NS


def _capacity(tokens_per_batch: int) -> int:
    cap = math.floor(_TOP_K * _CAP_FACTOR * tokens_per_batch / _N_EXP)
    cap += cap % 2
    return int(max(cap, _MIN_CAPACITY))


def _plan(tidx, a, ri, live, cap):
    rows_per_tok = _N_EXP * cap // 512
    kept = ri < cap
    row = tidx * rows_per_tok + a * (cap // 512) + jnp.where(kept, ri >> 9, 0)
    t = tidx.shape[0]
    jidx = lax.broadcasted_iota(jnp.int32, (t, 128), 1)
    shift = ((ri >> 7) & 3) << 3
    on = kept & live
    pat = jnp.where(on & (jidx == (ri & 127)), 1 << shift, 0)
    return row, pat.astype(jnp.int32)


def _routing_body(x_ref, wg_ref, w_ref, r_ref, uc_ref,
                  mi0_ref, mp0_ref, mi1_ref, mp1_ref, *, cap):
    x = x_ref[...]
    wg = wg_ref[...]
    logits = lax.dot_general(
        x, wg, (((1,), (1,)), ((), ())),
        preferred_element_type=jnp.float32)
    t, e = logits.shape
    idx = lax.broadcasted_iota(jnp.int32, (t, e), 1)

    m1 = jnp.max(logits, axis=1, keepdims=True)
    a1 = jnp.min(jnp.where(logits == m1, idx, e), axis=1, keepdims=True)
    oh1 = idx == a1
    masked = jnp.where(oh1, -jnp.inf, logits)
    m2 = jnp.max(masked, axis=1, keepdims=True)
    a2 = jnp.min(jnp.where(masked == m2, idx, e), axis=1, keepdims=True)
    oh2 = idx == a2

    d = jnp.exp(m2 - m1)
    p1 = 1.0 / (1.0 + d)
    p2 = d / (1.0 + d)

    oh1f = oh1.astype(jnp.float32)
    oh2f = oh2.astype(jnp.float32)
    row = lax.broadcasted_iota(jnp.int32, (t, t), 0)
    col = lax.broadcasted_iota(jnp.int32, (t, t), 1)
    stri = (col < row).astype(jnp.float32)
    oh12 = jnp.concatenate([oh1f, oh2f], axis=1)
    excl = lax.dot_general(
        stri, oh12, (((1,), (0,)), ((), ())),
        preferred_element_type=jnp.float32)
    tot1 = jnp.sum(oh1f, axis=0, keepdims=True)
    tot2 = jnp.sum(oh2f, axis=0, keepdims=True)
    rank1 = excl[:, :e]
    rank2 = tot1 + excl[:, e:]

    capf = jnp.float32(cap)
    w = (jnp.where(oh1 & (rank1 < capf), p1, 0.0)
         + jnp.where(oh2 & (rank2 < capf), p2, 0.0))
    rsel = jnp.where(oh1, rank1, jnp.where(oh2, rank2, 0.0))
    w_ref[...] = w
    r_ref[...] = rsel.astype(jnp.int32)
    uc_ref[...] = jnp.minimum(tot1 + tot2, capf).astype(jnp.int32)

    tidx = lax.broadcasted_iota(jnp.int32, (t, 1), 0)
    r1i = jnp.sum(jnp.where(oh1, rank1, 0.0), axis=1,
                  keepdims=True).astype(jnp.int32)
    r2i = jnp.sum(jnp.where(oh2, rank2, 0.0), axis=1,
                  keepdims=True).astype(jnp.int32)
    mi0, mp0 = _plan(tidx, a1, r1i, p1 != 0.0, cap)
    mi1, mp1 = _plan(tidx, a2, r2i, p2 != 0.0, cap)
    mi0_ref[...] = mi0
    mp0_ref[...] = mp0
    mi1_ref[...] = mi1
    mp1_ref[...] = mp1


def _build_body(w_ref, r_ref, cb_ref):
    tb, e = w_ref.shape
    c = cb_ref.shape[-1]
    w = w_ref[...][:, :, None]
    r = r_ref[...][:, :, None]
    cidx = lax.broadcasted_iota(jnp.int32, (tb, e, c), 2)
    cb_ref[...] = jnp.where(cidx == r, w, 0.0)


def _sc_mask_body(mi0_hbm, mp0_hbm, mi1_hbm, mp1_hbm, out_hbm,
                  zbuf, idx_v, val_v, sem, sem2, *, rows, ents):
    rows_per_w = rows // _NW
    ents_per_w = ents // _NW
    wid = lax.axis_index("s") * _NC + lax.axis_index("c")

    z = jnp.zeros((4, 16), jnp.int8)
    for j in range(zbuf.shape[0] // 4):
        for jj in range(zbuf.shape[1] // 16):
            zbuf[pl.ds(j * 4, 4), pl.ds(jj * 16, 16)] = z

    row0 = wid * rows_per_w
    zrows = zbuf.shape[0]
    copies = [
        pltpu.async_copy(zbuf, out_hbm.at[pl.ds(row0 + j * zrows, zrows)],
                         sem)
        for j in range(rows_per_w // zrows)
    ]
    for cpy in copies:
        cpy.wait()

    out32 = out_hbm.bitcast(jnp.int32)
    base = wid * ents_per_w
    for mi_hbm, mp_hbm in ((mi0_hbm, mp0_hbm), (mi1_hbm, mp1_hbm)):
        pltpu.sync_copy(mi_hbm.at[pl.ds(base, ents_per_w)], idx_v)
        pltpu.sync_copy(mp_hbm.at[pl.ds(base, ents_per_w)], val_v)
        pltpu.async_copy(val_v, out32.at[idx_v], sem2).wait()


def kernel(x, W_g):
    b, t, d = x.shape
    n = b * t
    e = W_g.shape[0]
    cap = _capacity(n)
    x2 = x.reshape(n, d)

    w_full, r_full, uc, mi0, mp0, mi1, mp1 = pl.pallas_call(
        functools.partial(_routing_body, cap=cap),
        out_shape=[
            jax.ShapeDtypeStruct((n, e), jnp.float32),
            jax.ShapeDtypeStruct((n, e), jnp.int32),
            jax.ShapeDtypeStruct((1, e), jnp.int32),
            jax.ShapeDtypeStruct((n, 1), jnp.int32),
            jax.ShapeDtypeStruct((n, 128), jnp.int32),
            jax.ShapeDtypeStruct((n, 1), jnp.int32),
            jax.ShapeDtypeStruct((n, 128), jnp.int32),
        ],
    )(x2, W_g)

    tb = 256
    cb = pl.pallas_call(
        _build_body,
        grid=(n // tb,),
        in_specs=[
            pl.BlockSpec((tb, e), lambda i: (i, 0)),
            pl.BlockSpec((tb, e), lambda i: (i, 0)),
        ],
        out_specs=pl.BlockSpec((tb, e, cap), lambda i: (i, 0, 0)),
        out_shape=jax.ShapeDtypeStruct((n, e, cap), jnp.float32),
    )(w_full, r_full)

    rows = n * e * cap // 128
    ents = n
    mesh = plsc.VectorSubcoreMesh(
        core_axis_name="c", subcore_axis_name="s",
        num_cores=_NC, num_subcores=_NS)
    sc_mask = pl.kernel(
        functools.partial(_sc_mask_body, rows=rows, ents=ents),
        out_type=jax.ShapeDtypeStruct((rows, 128), jnp.int8),
        mesh=mesh,
        scratch_types=[
            pltpu.VMEM((128, 128), jnp.int8),
            pltpu.VMEM((ents // _NW,), jnp.int32),
            pltpu.VMEM((ents // _NW, 128), jnp.int32),
            pltpu.SemaphoreType.DMA,
            pltpu.SemaphoreType.DMA,
        ],
    )
    mask_i8 = sc_mask(mi0.reshape(n), mp0, mi1.reshape(n), mp1)
    mask = mask_i8.view(jnp.bool_).reshape(n, e, cap)

    return uc.reshape(e), cb, mask


# speedup vs baseline: 1.0041x; 1.0041x over previous
import functools
import math

import jax
import jax.numpy as jnp
from jax import lax
from jax.experimental import pallas as pl
from jax.experimental.pallas import tpu as pltpu
from jax.experimental.pallas import tpu_sc as plsc

_N_EXP = 8
_TOP_K = 2
_CAP_FACTOR = 2.0
_MIN_CAPACITY = 4

_NC = 2
_NS = 16
_NW = _NC * _NS


def _capacity(tokens_per_batch: int) -> int:
    cap = math.floor(_TOP_K * _CAP_FACTOR * tokens_per_batch / _N_EXP)
    cap += cap % 2
    return int(max(cap, _MIN_CAPACITY))


def _plan(tidx, a, ri, live, cap):
    rows_per_tok = _N_EXP * cap // 512
    kept = ri < cap
    row = tidx * rows_per_tok + a * (cap // 512) + jnp.where(kept, ri >> 9, 0)
    t = tidx.shape[0]
    jidx = lax.broadcasted_iota(jnp.int32, (t, 128), 1)
    shift = ((ri >> 7) & 3) << 3
    on = kept & live
    pat = jnp.where(on & (jidx == (ri & 127)), 1 << shift, 0)
    return row, pat.astype(jnp.int32)


def _routing_body(x_ref, wg_ref, w_ref, r_ref, uc_ref,
                  mi0_ref, mp0_ref, mi1_ref, mp1_ref, *, cap):
    x = x_ref[...]
    wg = wg_ref[...]
    logits = lax.dot_general(
        x, wg, (((1,), (1,)), ((), ())),
        preferred_element_type=jnp.float32)
    t, e = logits.shape
    idx = lax.broadcasted_iota(jnp.int32, (t, e), 1)

    m1 = jnp.max(logits, axis=1, keepdims=True)
    a1 = jnp.min(jnp.where(logits == m1, idx, e), axis=1, keepdims=True)
    oh1 = idx == a1
    masked = jnp.where(oh1, -jnp.inf, logits)
    m2 = jnp.max(masked, axis=1, keepdims=True)
    a2 = jnp.min(jnp.where(masked == m2, idx, e), axis=1, keepdims=True)
    oh2 = idx == a2

    d = jnp.exp(m2 - m1)
    p1 = 1.0 / (1.0 + d)
    p2 = d / (1.0 + d)

    oh1f = oh1.astype(jnp.float32)
    oh2f = oh2.astype(jnp.float32)
    row = lax.broadcasted_iota(jnp.int32, (t, t), 0)
    col = lax.broadcasted_iota(jnp.int32, (t, t), 1)
    stri = (col < row).astype(jnp.float32)
    oh12 = jnp.concatenate([oh1f, oh2f], axis=1)
    excl = lax.dot_general(
        stri, oh12, (((1,), (0,)), ((), ())),
        preferred_element_type=jnp.float32)
    tot1 = jnp.sum(oh1f, axis=0, keepdims=True)
    tot2 = jnp.sum(oh2f, axis=0, keepdims=True)
    rank1 = excl[:, :e]
    rank2 = tot1 + excl[:, e:]

    capf = jnp.float32(cap)
    w = (jnp.where(oh1 & (rank1 < capf), p1, 0.0)
         + jnp.where(oh2 & (rank2 < capf), p2, 0.0))
    rsel = jnp.where(oh1, rank1, jnp.where(oh2, rank2, 0.0))
    w_ref[...] = w
    r_ref[...] = rsel.astype(jnp.int32)
    uc_ref[...] = jnp.minimum(tot1 + tot2, capf).astype(jnp.int32)

    tidx = lax.broadcasted_iota(jnp.int32, (t, 1), 0)
    r1i = jnp.sum(jnp.where(oh1, rank1, 0.0), axis=1,
                  keepdims=True).astype(jnp.int32)
    r2i = jnp.sum(jnp.where(oh2, rank2, 0.0), axis=1,
                  keepdims=True).astype(jnp.int32)
    mi0, mp0 = _plan(tidx, a1, r1i, p1 != 0.0, cap)
    mi1, mp1 = _plan(tidx, a2, r2i, p2 != 0.0, cap)
    mi0_ref[...] = mi0
    mp0_ref[...] = mp0
    mi1_ref[...] = mi1
    mp1_ref[...] = mp1


def _build_body(w_ref, r_ref, cb_ref):
    tb, e = w_ref.shape
    c = cb_ref.shape[-1]
    w = w_ref[...][:, :, None]
    r = r_ref[...][:, :, None]
    cidx = lax.broadcasted_iota(jnp.int32, (tb, e, c), 2)
    cb_ref[...] = jnp.where(cidx == r, w, 0.0)


def _sc_mask_body(mi0_hbm, mp0_hbm, mi1_hbm, mp1_hbm, out_hbm,
                  zbuf, idx_v, val_v, sem, sem2, *, rows, ents):
    rows_per_w = rows // _NW
    ents_per_w = ents // _NW
    wid = lax.axis_index("s") * _NC + lax.axis_index("c")

    z = jnp.zeros((4, 16), jnp.int8)
    for j in range(zbuf.shape[0] // 4):
        for jj in range(zbuf.shape[1] // 16):
            zbuf[pl.ds(j * 4, 4), pl.ds(jj * 16, 16)] = z

    row0 = wid * rows_per_w
    zrows = zbuf.shape[0]
    copies = [
        pltpu.async_copy(zbuf, out_hbm.at[pl.ds(row0 + j * zrows, zrows)],
                         sem)
        for j in range(rows_per_w // zrows)
    ]
    for cpy in copies:
        cpy.wait()

    out32 = out_hbm.bitcast(jnp.int32)
    base = wid * ents_per_w
    for mi_hbm, mp_hbm in ((mi0_hbm, mp0_hbm), (mi1_hbm, mp1_hbm)):
        pltpu.sync_copy(mi_hbm.at[pl.ds(base, ents_per_w)], idx_v)
        pltpu.sync_copy(mp_hbm.at[pl.ds(base, ents_per_w)], val_v)
        pltpu.async_copy(val_v, out32.at[idx_v], sem2).wait()


def kernel(x, W_g):
    b, t, d = x.shape
    n = b * t
    e = W_g.shape[0]
    cap = _capacity(n)
    x2 = x.reshape(n, d)

    w_full, r_full, uc, mi0, mp0, mi1, mp1 = pl.pallas_call(
        functools.partial(_routing_body, cap=cap),
        out_shape=[
            jax.ShapeDtypeStruct((n, e), jnp.float32),
            jax.ShapeDtypeStruct((n, e), jnp.int32),
            jax.ShapeDtypeStruct((1, e), jnp.int32),
            jax.ShapeDtypeStruct((n, 1), jnp.int32),
            jax.ShapeDtypeStruct((n, 128), jnp.int32),
            jax.ShapeDtypeStruct((n, 1), jnp.int32),
            jax.ShapeDtypeStruct((n, 128), jnp.int32),
        ],
    )(x2, W_g)

    tb = 256
    cb = pl.pallas_call(
        _build_body,
        grid=(n // tb,),
        in_specs=[
            pl.BlockSpec((tb, e), lambda i: (i, 0)),
            pl.BlockSpec((tb, e), lambda i: (i, 0)),
        ],
        out_specs=pl.BlockSpec((tb, e, cap), lambda i: (i, 0, 0)),
        out_shape=jax.ShapeDtypeStruct((n, e, cap), jnp.float32),
    )(w_full, r_full)

    rows = n * e * cap // 128
    ents = n
    mesh = plsc.VectorSubcoreMesh(
        core_axis_name="c", subcore_axis_name="s",
        num_cores=_NC, num_subcores=_NS)
    sc_mask = pl.kernel(
        functools.partial(_sc_mask_body, rows=rows, ents=ents),
        out_type=jax.ShapeDtypeStruct((rows, 128), jnp.int8),
        mesh=mesh,
        scratch_types=[
            pltpu.VMEM((1024, 128), jnp.int8),
            pltpu.VMEM((ents // _NW,), jnp.int32),
            pltpu.VMEM((ents // _NW, 128), jnp.int32),
            pltpu.SemaphoreType.DMA,
            pltpu.SemaphoreType.DMA,
        ],
    )
    mask_i8 = sc_mask(mi0.reshape(n), mp0, mi1.reshape(n), mp1)
    mask = mask_i8.view(jnp.bool_).reshape(n, e, cap)

    return uc.reshape(e), cb, mask


# speedup vs baseline: 2.1106x; 2.1020x over previous
import functools
import math

import jax
import jax.numpy as jnp
from jax import lax
from jax.experimental import pallas as pl

_N_EXP = 8
_TOP_K = 2
_CAP_FACTOR = 2.0
_MIN_CAPACITY = 4


def _capacity(tokens_per_batch: int) -> int:
    cap = math.floor(_TOP_K * _CAP_FACTOR * tokens_per_batch / _N_EXP)
    cap += cap % 2
    return int(max(cap, _MIN_CAPACITY))


def _routing_body(x_ref, wg_ref, w_ref, r_ref, uc_ref, *, cap):
    x = x_ref[...]
    wg = wg_ref[...]
    logits = lax.dot_general(
        x, wg, (((1,), (1,)), ((), ())),
        preferred_element_type=jnp.float32)
    t, e = logits.shape
    idx = lax.broadcasted_iota(jnp.int32, (t, e), 1)

    m1 = jnp.max(logits, axis=1, keepdims=True)
    a1 = jnp.min(jnp.where(logits == m1, idx, e), axis=1, keepdims=True)
    oh1 = idx == a1
    masked = jnp.where(oh1, -jnp.inf, logits)
    m2 = jnp.max(masked, axis=1, keepdims=True)
    a2 = jnp.min(jnp.where(masked == m2, idx, e), axis=1, keepdims=True)
    oh2 = idx == a2

    d = jnp.exp(m2 - m1)
    p1 = 1.0 / (1.0 + d)
    p2 = d / (1.0 + d)

    oh1f = oh1.astype(jnp.float32)
    oh2f = oh2.astype(jnp.float32)
    row = lax.broadcasted_iota(jnp.int32, (t, t), 0)
    col = lax.broadcasted_iota(jnp.int32, (t, t), 1)
    stri = (col < row).astype(jnp.float32)
    oh12 = jnp.concatenate([oh1f, oh2f], axis=1)
    excl = lax.dot_general(
        stri, oh12, (((1,), (0,)), ((), ())),
        preferred_element_type=jnp.float32)
    tot1 = jnp.sum(oh1f, axis=0, keepdims=True)
    tot2 = jnp.sum(oh2f, axis=0, keepdims=True)
    rank1 = excl[:, :e]
    rank2 = tot1 + excl[:, e:]

    capf = jnp.float32(cap)
    w = (jnp.where(oh1 & (rank1 < capf), p1, 0.0)
         + jnp.where(oh2 & (rank2 < capf), p2, 0.0))
    rsel = jnp.where(oh1, rank1, jnp.where(oh2, rank2, 0.0))
    w_ref[...] = w
    r_ref[...] = rsel.astype(jnp.int32)
    uc_ref[...] = jnp.minimum(tot1 + tot2, capf).astype(jnp.int32)


def _build_body(w_ref, r_ref, cb_ref, mask_ref):
    tb, e = w_ref.shape
    c = cb_ref.shape[-1]
    w = w_ref[...][:, :, None]
    r = r_ref[...][:, :, None]
    cidx = lax.broadcasted_iota(jnp.int32, (tb, e, c), 2)
    cb = jnp.where(cidx == r, w, 0.0)
    cb_ref[...] = cb
    mask_ref[...] = (cb != 0.0).astype(jnp.int8)


def kernel(x, W_g):
    b, t, d = x.shape
    n = b * t
    e = W_g.shape[0]
    cap = _capacity(n)
    x2 = x.reshape(n, d)

    w_full, r_full, uc = pl.pallas_call(
        functools.partial(_routing_body, cap=cap),
        out_shape=[
            jax.ShapeDtypeStruct((n, e), jnp.float32),
            jax.ShapeDtypeStruct((n, e), jnp.int32),
            jax.ShapeDtypeStruct((1, e), jnp.int32),
        ],
    )(x2, W_g)

    tb = 256
    cb, mask_i8 = pl.pallas_call(
        _build_body,
        grid=(n // tb,),
        in_specs=[
            pl.BlockSpec((tb, e), lambda i: (i, 0)),
            pl.BlockSpec((tb, e), lambda i: (i, 0)),
        ],
        out_specs=[
            pl.BlockSpec((tb, e, cap), lambda i: (i, 0, 0)),
            pl.BlockSpec((tb, e, cap), lambda i: (i, 0, 0)),
        ],
        out_shape=[
            jax.ShapeDtypeStruct((n, e, cap), jnp.float32),
            jax.ShapeDtypeStruct((n, e, cap), jnp.int8),
        ],
    )(w_full, r_full)
    mask = mask_i8.view(jnp.bool_)

    return uc.reshape(e), cb, mask
